# Initial kernel scaffold; baseline (speedup 1.0000x reference)
import jax
import jax.numpy as jnp
from jax.experimental import pallas as pl


def kernel(P, d_hw, circuit_edge_pairs, circuit_edge_weights):
    raise NotImplementedError("write your pallas kernel here")


# speedup vs baseline: 5.9052x; 5.9052x over previous
import functools

import jax
import jax.numpy as jnp
from jax import lax
from jax.experimental import pallas as pl
from jax.experimental.pallas import tpu as pltpu
from jax.experimental.pallas import tpu_sc as plsc


def _pdp_body(d_ref, p_ref, m_ref):
    dsw = 3.0 * jnp.maximum(d_ref[...] - 1.0, 0.0)
    p = p_ref[0]
    t1 = lax.dot(p, dsw, preferred_element_type=jnp.float32)
    m_ref[0] = lax.dot_general(t1, p, (((1,), (1,)), ((), ())),
                               preferred_element_type=jnp.float32)


def _compute_m(P, d_hw):
    B, N, _ = P.shape
    return pl.pallas_call(
        _pdp_body,
        grid=(B,),
        in_specs=[
            pl.BlockSpec((N, N), lambda b: (0, 0)),
            pl.BlockSpec((1, N, N), lambda b: (b, 0, 0)),
        ],
        out_specs=pl.BlockSpec((1, N, N), lambda b: (b, 0, 0)),
        out_shape=jax.ShapeDtypeStruct((B, N, N), jnp.float32),
    )(d_hw, P)


def _sc_gather_dot(gidx3, w3, m_flat, B, L=16):
    NW, K, C = gidx3.shape
    mesh = plsc.VectorSubcoreMesh(core_axis_name="c", subcore_axis_name="s")

    @functools.partial(
        pl.kernel,
        mesh=mesh,
        out_type=(
            jax.ShapeDtypeStruct((B, 2 * L), jnp.float32),
            jax.ShapeDtypeStruct((B, 2 * L), jnp.float32),
        ),
        scratch_types=[
            pltpu.VMEM((K, C), jnp.int32),
            pltpu.VMEM((K, C), jnp.float32),
            pltpu.VMEM((K, C), jnp.float32),
            pltpu.VMEM((L,), jnp.float32),
            pltpu.VMEM((L,), jnp.float32),
        ],
    )
    def sc_kernel(gidx_hbm, w_hbm, m_hbm, num_hbm, den_hbm,
                  idx_v, w_v, vals_v, num_v, den_v):
        c = lax.axis_index("c")
        s = lax.axis_index("s")
        wid = c * 16 + s
        b = wid // 2
        half = wid % 2
        pltpu.sync_copy(gidx_hbm.at[wid], idx_v)
        pltpu.sync_copy(w_hbm.at[wid], w_v)
        for k in range(K):
            pltpu.sync_copy(m_hbm.at[idx_v.at[k]], vals_v.at[k])
        num_v[...] = jnp.zeros((L,), jnp.float32)
        den_v[...] = jnp.zeros((L,), jnp.float32)
        for k in range(K):
            wk = w_v.at[k]
            vk = vals_v.at[k]

            @pl.loop(0, C // L)
            def _(i):
                sl = pl.ds(i * L, L)
                wv = wk[sl]
                num_v[...] += vk[sl] * wv
                den_v[...] += wv

        pltpu.sync_copy(num_v, num_hbm.at[b, pl.ds(half * L, L)])
        pltpu.sync_copy(den_v, den_hbm.at[b, pl.ds(half * L, L)])

    return sc_kernel(gidx3, w3, m_flat)


def _final_body(num_ref, den_ref, out_ref):
    B = num_ref.shape[0]
    num = jnp.sum(num_ref[...], axis=1)
    den = jnp.sum(den_ref[...], axis=1)
    out_ref[0, 0] = jnp.sum(num / jnp.maximum(den, 1e-8)) / B


def _finalize(num_parts, den_parts):
    return pl.pallas_call(
        _final_body,
        out_specs=pl.BlockSpec(memory_space=pltpu.SMEM),
        out_shape=jax.ShapeDtypeStruct((1, 1), jnp.float32),
    )(num_parts, den_parts)


def kernel(P, d_hw, circuit_edge_pairs, circuit_edge_weights):
    B, N, _ = P.shape
    _, E, _ = circuit_edge_pairs.shape
    NW = 32
    per_w = (B * E) // NW
    K, C = per_w // 128, 128

    pairs = circuit_edge_pairs.astype(jnp.int32)
    gidx = (jnp.arange(B, dtype=jnp.int32)[:, None] * (N * N)
            + pairs[..., 0] * N + pairs[..., 1])
    gidx3 = gidx.reshape(NW, K, C)
    w3 = circuit_edge_weights.reshape(NW, K, C)

    M = _compute_m(P, d_hw)
    num_parts, den_parts = _sc_gather_dot(gidx3, w3, M.reshape(B * N * N), B)
    out = _finalize(num_parts, den_parts)
    return out[0, 0]
